# baseline (speedup 1.0000x reference)
import jax
import jax.numpy as jnp
from jax.experimental import pallas as pl

D = 512
NW = 20
NO = 10
BATCH = 1024


def _body(feats4_ref, word_ref, obj_ref, W_ow_ref, a_ow_s_ref, a_ow_d_ref, b_ow_ref,
          W1_ref, b1_ref, W2_ref, b2_ref,
          Wm1_ref, bm1_ref, Wm2_ref, bm2_ref, out_ref):
    f32 = jnp.float32
    bf16 = jnp.bfloat16

    wf0 = word_ref[...].astype(bf16)
    of0 = obj_ref[...].astype(bf16)
    W_ow = W_ow_ref[...]
    b_ow = b_ow_ref[...]

    Hw = jnp.dot(wf0, W_ow, preferred_element_type=f32)
    Ho = jnp.dot(of0, W_ow, preferred_element_type=f32)

    uw = jnp.zeros((NO, D), f32)
    for h in range(2):
        Hwh = Hw[:, h * D:(h + 1) * D]
        ad_col = jax.lax.dot_general(
            Ho[:, h * D:(h + 1) * D], a_ow_d_ref[h:h + 1, :],
            dimension_numbers=(((1,), (1,)), ((), ())),
            precision=jax.lax.Precision.HIGHEST,
            preferred_element_type=f32)
        as_row = jax.lax.dot_general(
            a_ow_s_ref[h:h + 1, :], Hwh,
            dimension_numbers=(((1,), (1,)), ((), ())),
            precision=jax.lax.Precision.HIGHEST,
            preferred_element_type=f32)

        alpha = ad_col + as_row
        alpha = jnp.where(alpha > 0, alpha, 0.2 * alpha)
        amax = jnp.max(alpha, axis=1, keepdims=True)
        ex = jnp.exp(alpha - amax)
        denom = jnp.sum(ex, axis=1, keepdims=True) + 1e-16
        coef = ex / denom
        uw = uw + jnp.dot(coef, Hwh,
                          precision=jax.lax.Precision.HIGHEST,
                          preferred_element_type=f32)
    uw = 0.5 * uw + b_ow

    uw16 = uw.astype(bf16).astype(f32)
    bow16 = b_ow.astype(bf16).astype(f32)
    s_col = jnp.sum(uw16 * bow16, axis=1, keepdims=True)
    s_max = jnp.max(s_col, axis=0, keepdims=True)
    s_ex = jnp.exp(s_col - s_max)
    attn = s_ex / jnp.sum(s_ex, axis=0, keepdims=True)
    attn16 = attn.astype(bf16).astype(f32)
    aligned = jnp.sum(attn16 * uw16, axis=0, keepdims=True)

    X5 = jnp.concatenate([feats4_ref[...], aligned], axis=0)
    row_sum = jnp.sum(X5, axis=1, keepdims=True)
    ind = (row_sum != 0).astype(f32)
    sq = jnp.sum(X5 * X5, axis=1, keepdims=True) + ind
    nrm = jnp.maximum(jnp.sqrt(sq), 1e-12)
    Xn = X5 / nrm
    indn = ind / nrm

    W1b = W1_ref[...]
    H1 = jnp.dot(Xn.astype(bf16), W1b[:D, :], preferred_element_type=f32)
    H1 = H1 + (indn.astype(bf16).astype(f32)
               * W1b[D:D + 1, :].astype(f32)) + b1_ref[...]
    H1 = jnp.maximum(H1, 0.0)
    H2 = jnp.dot(H1.astype(bf16), W2_ref[...],
                 preferred_element_type=f32) + b2_ref[...]

    z = jnp.zeros((1, D), f32)
    Wm1_sum = jnp.zeros((D, D), f32)
    H2b = H2.astype(bf16)
    for k in range(5):
        Wm1_k = Wm1_ref[k * D:(k + 1) * D, :]
        z = z + jnp.dot(H2b[k:k + 1, :], Wm1_k, preferred_element_type=f32)
        Wm1_sum = Wm1_sum + Wm1_k.astype(f32)
    zc = jnp.dot(b2_ref[...].astype(bf16), Wm1_sum.astype(bf16),
                 preferred_element_type=f32)

    Wm2b = Wm2_ref[...]
    m0 = jnp.dot(jnp.maximum(z + bm1_ref[...], 0.0).astype(bf16), Wm2b,
                 preferred_element_type=f32) + bm2_ref[...]
    mc = jnp.dot(jnp.maximum(zc + bm1_ref[...], 0.0).astype(bf16), Wm2b,
                 preferred_element_type=f32) + bm2_ref[...]

    rows = jax.lax.broadcasted_iota(jnp.int32, (BATCH, 3), 0)
    out_ref[...] = jnp.where(rows == 0,
                             jnp.broadcast_to(m0, (BATCH, 3)),
                             jnp.broadcast_to(mc, (BATCH, 3)))


def kernel(text_feat, image_feat, imgtxt_feat, face_feat, word_feat, object_feat,
           W_ow, a_ow_s, a_ow_d, b_ow, W1, a1s, a1d, b1, W2, a2s, a2d, b2,
           Wm1, bm1, Wm2, bm2):
    bf16 = jnp.bfloat16
    feats4 = jnp.stack([text_feat[0], image_feat[0],
                        imgtxt_feat[0], face_feat[0]], axis=0)
    return pl.pallas_call(
        _body,
        out_shape=jax.ShapeDtypeStruct((BATCH, 3), jnp.float32),
    )(feats4, word_feat[0], object_feat[0],
      W_ow.astype(bf16), a_ow_s, a_ow_d, b_ow[None, :],
      W1.astype(bf16), b1[None, :], W2.astype(bf16), b2[None, :],
      Wm1.astype(bf16), bm1[None, :], Wm2.astype(bf16), bm2[None, :])


# speedup vs baseline: 1.3613x; 1.3613x over previous
import jax
import jax.numpy as jnp
from jax.experimental import pallas as pl

D = 512
NW = 20
NO = 10
BATCH = 1024


def _body(feats4_ref, word_ref, obj_ref, W_ow_ref, a_ow_s_ref, a_ow_d_ref, b_ow_ref,
          W1_ref, b1_ref, W2_ref, b2_ref,
          Wm1_ref, bm1_ref, Wm2_ref, bm2_ref, out_ref):
    f32 = jnp.float32
    bf16 = jnp.bfloat16

    wf0 = word_ref[...].astype(bf16)
    of0 = obj_ref[...].astype(bf16)
    W_ow = W_ow_ref[...].astype(bf16)
    b_ow = b_ow_ref[...]

    Hw = jnp.dot(wf0, W_ow, preferred_element_type=f32)
    Ho = jnp.dot(of0, W_ow, preferred_element_type=f32)

    uw = jnp.zeros((NO, D), f32)
    for h in range(2):
        Hwh = Hw[:, h * D:(h + 1) * D]
        ad_col = jax.lax.dot_general(
            Ho[:, h * D:(h + 1) * D], a_ow_d_ref[h:h + 1, :],
            dimension_numbers=(((1,), (1,)), ((), ())),
            precision=jax.lax.Precision.HIGHEST,
            preferred_element_type=f32)
        as_row = jax.lax.dot_general(
            a_ow_s_ref[h:h + 1, :], Hwh,
            dimension_numbers=(((1,), (1,)), ((), ())),
            precision=jax.lax.Precision.HIGHEST,
            preferred_element_type=f32)

        alpha = ad_col + as_row
        alpha = jnp.where(alpha > 0, alpha, 0.2 * alpha)
        amax = jnp.max(alpha, axis=1, keepdims=True)
        ex = jnp.exp(alpha - amax)
        denom = jnp.sum(ex, axis=1, keepdims=True) + 1e-16
        coef = ex / denom
        uw = uw + jnp.dot(coef, Hwh,
                          precision=jax.lax.Precision.HIGHEST,
                          preferred_element_type=f32)
    uw = 0.5 * uw + b_ow

    uw16 = uw.astype(bf16).astype(f32)
    bow16 = b_ow.astype(bf16).astype(f32)
    s_col = jnp.sum(uw16 * bow16, axis=1, keepdims=True)
    s_max = jnp.max(s_col, axis=0, keepdims=True)
    s_ex = jnp.exp(s_col - s_max)
    attn = s_ex / jnp.sum(s_ex, axis=0, keepdims=True)
    attn16 = attn.astype(bf16).astype(f32)
    aligned = jnp.sum(attn16 * uw16, axis=0, keepdims=True)

    X5 = jnp.concatenate([feats4_ref[...], aligned], axis=0)
    row_sum = jnp.sum(X5, axis=1, keepdims=True)
    ind = (row_sum != 0).astype(f32)
    sq = jnp.sum(X5 * X5, axis=1, keepdims=True) + ind
    nrm = jnp.maximum(jnp.sqrt(sq), 1e-12)
    Xn = X5 / nrm
    indn = ind / nrm

    W1b = W1_ref[...].astype(bf16)
    H1 = jnp.dot(Xn.astype(bf16), W1b[:D, :], preferred_element_type=f32)
    H1 = H1 + (indn.astype(bf16).astype(f32)
               * W1b[D:D + 1, :].astype(f32)) + b1_ref[...]
    H1 = jnp.maximum(H1, 0.0)
    H2 = jnp.dot(H1.astype(bf16), W2_ref[...].astype(bf16),
                 preferred_element_type=f32) + b2_ref[...]

    z = jnp.zeros((1, D), f32)
    Wm1_sum = jnp.zeros((D, D), f32)
    H2b = H2.astype(bf16)
    for k in range(5):
        Wm1_k = Wm1_ref[k * D:(k + 1) * D, :].astype(bf16)
        z = z + jnp.dot(H2b[k:k + 1, :], Wm1_k, preferred_element_type=f32)
        Wm1_sum = Wm1_sum + Wm1_k.astype(f32)
    zc = jnp.dot(b2_ref[...].astype(bf16), Wm1_sum.astype(bf16),
                 preferred_element_type=f32)

    Wm2b = Wm2_ref[...].astype(bf16)
    m0 = jnp.dot(jnp.maximum(z + bm1_ref[...], 0.0).astype(bf16), Wm2b,
                 preferred_element_type=f32) + bm2_ref[...]
    mc = jnp.dot(jnp.maximum(zc + bm1_ref[...], 0.0).astype(bf16), Wm2b,
                 preferred_element_type=f32) + bm2_ref[...]

    rows = jax.lax.broadcasted_iota(jnp.int32, (BATCH, 3), 0)
    out_ref[...] = jnp.where(rows == 0,
                             jnp.broadcast_to(m0, (BATCH, 3)),
                             jnp.broadcast_to(mc, (BATCH, 3)))


def kernel(text_feat, image_feat, imgtxt_feat, face_feat, word_feat, object_feat,
           W_ow, a_ow_s, a_ow_d, b_ow, W1, a1s, a1d, b1, W2, a2s, a2d, b2,
           Wm1, bm1, Wm2, bm2):
    feats4 = jnp.stack([text_feat[0], image_feat[0],
                        imgtxt_feat[0], face_feat[0]], axis=0)
    return pl.pallas_call(
        _body,
        out_shape=jax.ShapeDtypeStruct((BATCH, 3), jnp.float32),
    )(feats4, word_feat[0], object_feat[0],
      W_ow, a_ow_s, a_ow_d, b_ow[None, :],
      W1, b1[None, :], W2, b2[None, :],
      Wm1, bm1[None, :], Wm2, bm2[None, :])


# speedup vs baseline: 1.3678x; 1.0048x over previous
import jax
import jax.numpy as jnp
from jax.experimental import pallas as pl

D = 512
NW = 20
NO = 10
BATCH = 1024


def _body(feats4_ref, word_ref, obj_ref, W_ow_ref, a_ow_s_ref, a_ow_d_ref, b_ow_ref,
          W1_ref, b1_ref, W2_ref, b2_ref,
          Wm1_ref, bm1_ref, Wm2_ref, bm2_ref, out_ref):
    f32 = jnp.float32
    bf16 = jnp.bfloat16

    wf0 = word_ref[...].astype(bf16)
    of0 = obj_ref[...].astype(bf16)
    W_ow = W_ow_ref[...].astype(bf16)
    b_ow = b_ow_ref[...]

    Hw = jnp.dot(wf0, W_ow, preferred_element_type=f32)
    Ho = jnp.dot(of0, W_ow, preferred_element_type=f32)

    uw = jnp.zeros((NO, D), f32)
    for h in range(2):
        Hwh = Hw[:, h * D:(h + 1) * D]
        ad_col = jax.lax.dot_general(
            Ho[:, h * D:(h + 1) * D], a_ow_d_ref[h:h + 1, :],
            dimension_numbers=(((1,), (1,)), ((), ())),
            precision=jax.lax.Precision.HIGHEST,
            preferred_element_type=f32)
        as_row = jax.lax.dot_general(
            a_ow_s_ref[h:h + 1, :], Hwh,
            dimension_numbers=(((1,), (1,)), ((), ())),
            precision=jax.lax.Precision.HIGHEST,
            preferred_element_type=f32)

        alpha = ad_col + as_row
        alpha = jnp.where(alpha > 0, alpha, 0.2 * alpha)
        amax = jnp.max(alpha, axis=1, keepdims=True)
        ex = jnp.exp(alpha - amax)
        denom = jnp.sum(ex, axis=1, keepdims=True) + 1e-16
        coef = ex / denom
        uw = uw + jnp.dot(coef, Hwh,
                          precision=jax.lax.Precision.HIGHEST,
                          preferred_element_type=f32)
    uw = 0.5 * uw + b_ow

    uw16 = uw.astype(bf16).astype(f32)
    bow16 = b_ow.astype(bf16).astype(f32)
    s_col = jnp.sum(uw16 * bow16, axis=1, keepdims=True)
    s_max = jnp.max(s_col, axis=0, keepdims=True)
    s_ex = jnp.exp(s_col - s_max)
    attn = s_ex / jnp.sum(s_ex, axis=0, keepdims=True)
    attn16 = attn.astype(bf16).astype(f32)
    aligned = jnp.sum(attn16 * uw16, axis=0, keepdims=True)

    X5 = jnp.concatenate([feats4_ref[...], aligned], axis=0)
    row_sum = jnp.sum(X5, axis=1, keepdims=True)
    ind = (row_sum != 0).astype(f32)
    sq = jnp.sum(X5 * X5, axis=1, keepdims=True) + ind
    nrm = jnp.maximum(jnp.sqrt(sq), 1e-12)
    Xn = X5 / nrm
    indn = ind / nrm

    W1b = W1_ref[...].astype(bf16)
    H1 = jnp.dot(Xn.astype(bf16), W1b[:D, :], preferred_element_type=f32)
    H1 = H1 + (indn.astype(bf16).astype(f32)
               * W1b[D:D + 1, :].astype(f32)) + b1_ref[...]
    H1 = jnp.maximum(H1, 0.0)
    H2 = jnp.dot(H1.astype(bf16), W2_ref[...].astype(bf16),
                 preferred_element_type=f32) + b2_ref[...]

    z = jnp.zeros((1, D), f32)
    Wm1_sum = jnp.zeros((D, D), f32)
    H2b = H2.astype(bf16)
    for k in range(5):
        Wm1_k = Wm1_ref[k * D:(k + 1) * D, :].astype(bf16)
        z = z + jnp.dot(H2b[k:k + 1, :], Wm1_k, preferred_element_type=f32)
        Wm1_sum = Wm1_sum + Wm1_k.astype(f32)
    zc = jnp.dot(b2_ref[...].astype(bf16), Wm1_sum.astype(bf16),
                 preferred_element_type=f32)

    Wm2b = Wm2_ref[...].astype(bf16)
    m0 = jnp.dot(jnp.maximum(z + bm1_ref[...], 0.0).astype(bf16), Wm2b,
                 preferred_element_type=f32) + bm2_ref[...]
    mc = jnp.dot(jnp.maximum(zc + bm1_ref[...], 0.0).astype(bf16), Wm2b,
                 preferred_element_type=f32) + bm2_ref[...]

    out_ref[...] = jnp.broadcast_to(mc, (BATCH, 3))
    out_ref[0:1, :] = m0


def kernel(text_feat, image_feat, imgtxt_feat, face_feat, word_feat, object_feat,
           W_ow, a_ow_s, a_ow_d, b_ow, W1, a1s, a1d, b1, W2, a2s, a2d, b2,
           Wm1, bm1, Wm2, bm2):
    feats4 = jnp.stack([text_feat[0], image_feat[0],
                        imgtxt_feat[0], face_feat[0]], axis=0)
    return pl.pallas_call(
        _body,
        out_shape=jax.ShapeDtypeStruct((BATCH, 3), jnp.float32),
    )(feats4, word_feat[0], object_feat[0],
      W_ow, a_ow_s, a_ow_d, b_ow[None, :],
      W1, b1[None, :], W2, b2[None, :],
      Wm1, bm1[None, :], Wm2, bm2[None, :])
